# baseline (speedup 1.0000x reference)
import functools

import jax
import jax.numpy as jnp
from jax import lax
from jax.experimental import pallas as pl
from jax.experimental.pallas import tpu as pltpu
from jax.experimental.pallas import tpu_sc as plsc

_NUM_MATERIALS = 1000000
_EMBED_DIM = 64
_BATCH = 16384

_NC = 2
_NS = 16
_NW = _NC * _NS
_B_PER_W = _BATCH // _NW
_CHUNK = 128
_NCH = _B_PER_W // _CHUNK

_mesh = plsc.VectorSubcoreMesh(core_axis_name="c", subcore_axis_name="s")


@functools.partial(
    pl.kernel,
    mesh=_mesh,
    out_type=jax.ShapeDtypeStruct((_BATCH, _EMBED_DIM), jnp.float32),
    scratch_types=[
        pltpu.VMEM((_NCH, _CHUNK), jnp.int32),
        pltpu.VMEM((_B_PER_W, _EMBED_DIM), jnp.float32),
        pltpu.SemaphoreType.DMA,
    ],
    compiler_params=pltpu.CompilerParams(use_tc_tiling_on_sc=False),
)
def _gather_kernel(table_hbm, idx_hbm, out_hbm, idx_v, rows_v, sem):
    wid = lax.axis_index("s") * _NC + lax.axis_index("c")
    pltpu.sync_copy(idx_hbm.at[wid], idx_v)
    copies = [
        pltpu.async_copy(
            table_hbm.at[idx_v.at[j]],
            rows_v.at[pl.ds(j * _CHUNK, _CHUNK)],
            sem,
        )
        for j in range(_NCH)
    ]
    for c in copies:
        c.wait()
    pltpu.sync_copy(rows_v, out_hbm.at[pl.ds(wid * _B_PER_W, _B_PER_W)])


def kernel(embeddings, material_index):
    idx3 = material_index.reshape(_NW, _NCH, _CHUNK)
    return _gather_kernel(embeddings, idx3)


# speedup vs baseline: 1.6440x; 1.6440x over previous
import functools

import jax
import jax.numpy as jnp
from jax import lax
from jax.experimental import pallas as pl
from jax.experimental.pallas import tpu as pltpu
from jax.experimental.pallas import tpu_sc as plsc

_NUM_MATERIALS = 1000000
_EMBED_DIM = 64
_BATCH = 16384

_NC = 2
_NS = 16
_NW = _NC * _NS
_B_PER_W = _BATCH // _NW
_GRP = 16
_NGRP = _B_PER_W // _GRP

_mesh = plsc.VectorSubcoreMesh(core_axis_name="c", subcore_axis_name="s")


@functools.partial(
    pl.kernel,
    mesh=_mesh,
    out_type=jax.ShapeDtypeStruct((_BATCH, _EMBED_DIM), jnp.float32),
    scratch_types=[
        pltpu.VMEM((_B_PER_W,), jnp.int32),
        pltpu.VMEM((_B_PER_W, _EMBED_DIM), jnp.float32),
        pltpu.SemaphoreType.DMA,
    ],
)
def _gather_kernel(table_hbm, idx_hbm, out_hbm, idx_v, rows_v, sem):
    wid = lax.axis_index("s") * _NC + lax.axis_index("c")
    base = wid * _B_PER_W

    pltpu.sync_copy(idx_hbm.at[pl.ds(base, _B_PER_W)], idx_v)

    def group_body(g, carry):
        goff = pl.multiple_of(g * _GRP, _GRP)
        rvec = idx_v[pl.ds(goff, _GRP)]
        copies = [
            pltpu.make_async_copy(
                table_hbm.at[rvec[l]], rows_v.at[goff + l], sem
            )
            for l in range(_GRP)
        ]
        for c in copies:
            c.start()
        for c in copies:
            c.wait()
        return carry

    lax.fori_loop(0, _NGRP, group_body, 0)
    pltpu.sync_copy(rows_v, out_hbm.at[pl.ds(base, _B_PER_W)])


def kernel(embeddings, material_index):
    return _gather_kernel(embeddings, material_index)


# speedup vs baseline: 2.5808x; 1.5698x over previous
import functools

import jax
import jax.numpy as jnp
from jax import lax
from jax.experimental import pallas as pl
from jax.experimental.pallas import tpu as pltpu
from jax.experimental.pallas import tpu_sc as plsc

_NUM_MATERIALS = 1000000
_EMBED_DIM = 64
_BATCH = 16384

_NC = 2
_NS = 16
_NW = _NC * _NS
_B_PER_W = _BATCH // _NW
_GRP = 16
_NGRP = _B_PER_W // _GRP
_NPH = 2
_BPP = 8 // _NPH

_mesh = plsc.VectorSubcoreMesh(core_axis_name="c", subcore_axis_name="s")


@functools.partial(
    pl.kernel,
    mesh=_mesh,
    out_type=jax.ShapeDtypeStruct((_EMBED_DIM, _BATCH), jnp.float32),
    scratch_types=[
        pltpu.VMEM((_B_PER_W,), jnp.int32),
        pltpu.VMEM((_GRP, _BPP, 8, 128), jnp.float32),
        pltpu.VMEM((_EMBED_DIM, _B_PER_W), jnp.float32),
        pltpu.SemaphoreType.DMA,
    ],
    compiler_params=pltpu.CompilerParams(needs_layout_passes=False),
)
def _gather_kernel(table_hbm, idx_hbm, out_hbm, idx_v, buf_v, cols_v, sem):
    wid = lax.axis_index("s") * _NC + lax.axis_index("c")
    base = wid * _B_PER_W

    pltpu.sync_copy(idx_hbm.at[pl.ds(base, _B_PER_W)], idx_v)
    lane = lax.iota(jnp.int32, 16)

    def wave_body(g, carry):
        goff = pl.multiple_of(g * _GRP, _GRP)
        rvec = idx_v[pl.ds(goff, _GRP)]
        rm_vec = rvec & 127
        for phase in range(_NPH):
            copies = []
            for l in range(_GRP):
                rq = pl.multiple_of(rvec[l] & -128, 128)
                copies.append(
                    pltpu.make_async_copy(
                        table_hbm.at[
                            pl.ds(phase * _BPP, _BPP), :, pl.ds(rq, 128)
                        ],
                        buf_v.at[l],
                        sem,
                    )
                )
            for cp in copies:
                cp.start()
            for cp in copies:
                cp.wait()
            for cc in range(_BPP * 8):
                col = phase * _BPP * 8 + cc
                vals = plsc.load_gather(
                    buf_v,
                    [
                        lane,
                        jnp.full((16,), cc // 8, jnp.int32),
                        jnp.full((16,), cc % 8, jnp.int32),
                        rm_vec,
                    ],
                )
                cols_v[col, pl.ds(goff, _GRP)] = vals
        return carry

    lax.fori_loop(0, _NGRP, wave_body, 0)
    pltpu.sync_copy(cols_v, out_hbm.at[:, pl.ds(base, _B_PER_W)])


def kernel(embeddings, material_index):
    table3 = embeddings.T.reshape(8, 8, _NUM_MATERIALS)
    out_t = _gather_kernel(table3, material_index)
    return out_t.T


# speedup vs baseline: 3.2229x; 1.2488x over previous
import functools

import jax
import jax.numpy as jnp
from jax import lax
from jax.experimental import pallas as pl
from jax.experimental.pallas import tpu as pltpu
from jax.experimental.pallas import tpu_sc as plsc

_NUM_MATERIALS = 1000000
_EMBED_DIM = 64
_BATCH = 16384

_NC = 2
_NS = 16
_NW = _NC * _NS
_B_PER_W = _BATCH // _NW
_GRP = 16
_NGRP = _B_PER_W // _GRP
_NPH = 4
_BPP = 8 // _NPH
_NSTEP = _NGRP * _NPH

_mesh = plsc.VectorSubcoreMesh(core_axis_name="c", subcore_axis_name="s")


@functools.partial(
    pl.kernel,
    mesh=_mesh,
    out_type=jax.ShapeDtypeStruct((_EMBED_DIM, _BATCH), jnp.float32),
    scratch_types=[
        pltpu.VMEM((_B_PER_W,), jnp.int32),
        pltpu.VMEM((2, _GRP, _BPP, 8, 128), jnp.float32),
        pltpu.VMEM((_EMBED_DIM, _B_PER_W), jnp.float32),
        pltpu.SemaphoreType.DMA,
    ],
    compiler_params=pltpu.CompilerParams(needs_layout_passes=False),
)
def _gather_kernel(table_hbm, idx_hbm, out_hbm, idx_v, buf_v, cols_v, sem):
    wid = lax.axis_index("s") * _NC + lax.axis_index("c")
    base = wid * _B_PER_W

    pltpu.sync_copy(idx_hbm.at[pl.ds(base, _B_PER_W)], idx_v)
    lane = lax.iota(jnp.int32, 16)

    def step_copies(s):
        w = lax.shift_right_logical(s, 2)
        ph = s & (_NPH - 1)
        par = s & 1
        rvec = idx_v[pl.ds(pl.multiple_of(w * _GRP, _GRP), _GRP)]
        return [
            pltpu.make_async_copy(
                table_hbm.at[
                    pl.ds(ph * _BPP, _BPP),
                    :,
                    pl.ds(pl.multiple_of(rvec[l] & -128, 128), 128),
                ],
                buf_v.at[par, l],
                sem,
            )
            for l in range(_GRP)
        ]

    def fire(s):
        for cp in step_copies(s):
            cp.start()

    fire(0)

    def step_body(s, carry):
        @pl.when(s < _NSTEP - 1)
        def _():
            fire(s + 1)

        for cp in step_copies(s):
            cp.wait()

        w = lax.shift_right_logical(s, 2)
        ph = s & (_NPH - 1)
        par = s & 1
        goff = pl.multiple_of(w * _GRP, _GRP)
        rm_vec = idx_v[pl.ds(goff, _GRP)] & 127
        par_vec = jnp.full((16,), 0, jnp.int32) + par
        for cc in range(_BPP * 8):
            vals = plsc.load_gather(
                buf_v,
                [
                    par_vec,
                    lane,
                    jnp.full((16,), cc // 8, jnp.int32),
                    jnp.full((16,), cc % 8, jnp.int32),
                    rm_vec,
                ],
            )
            cols_v[ph * (_BPP * 8) + cc, pl.ds(goff, _GRP)] = vals
        return carry

    lax.fori_loop(0, _NSTEP, step_body, 0)
    pltpu.sync_copy(cols_v, out_hbm.at[:, pl.ds(base, _B_PER_W)])


def kernel(embeddings, material_index):
    table3 = embeddings.T.reshape(8, 8, _NUM_MATERIALS)
    out_t = _gather_kernel(table3, material_index)
    return out_t.T
